# baseline (speedup 1.0000x reference)
import functools

import jax
import jax.numpy as jnp
from jax.experimental import pallas as pl
from jax.experimental.pallas import tpu as pltpu

B, S, D = 16384, 26, 128
K = 5
IDX_LO = 1
BB = 512
NB = B // BB


def _body(x_ref, o_ref):
    o_ref[...] = x_ref[:, IDX_LO:IDX_LO + K, :]


@jax.jit
def kernel(x):
    return pl.pallas_call(
        _body,
        grid=(NB,),
        in_specs=[
            pl.BlockSpec((BB, S, D), lambda i: (i, 0, 0)),
        ],
        out_specs=pl.BlockSpec((BB, K, D), lambda i: (i, 0, 0)),
        out_shape=jax.ShapeDtypeStruct((B, K, D), jnp.float32),
    )(x)


# speedup vs baseline: 1.2059x; 1.2059x over previous
import functools

import jax
import jax.numpy as jnp
from jax.experimental import pallas as pl
from jax.experimental.pallas import tpu as pltpu

B, S, D = 16384, 26, 128
K = 5
IDX_LO = 1
BB = 512
NB = B // BB


def _body(x_ref, o_ref):
    o_ref[...] = x_ref[:, IDX_LO:IDX_LO + K, :]


@jax.jit
def kernel(x):
    return pl.pallas_call(
        _body,
        grid=(NB,),
        in_specs=[
            pl.BlockSpec((BB, 8, D), lambda i: (i, 0, 0)),
        ],
        out_specs=pl.BlockSpec((BB, K, D), lambda i: (i, 0, 0)),
        out_shape=jax.ShapeDtypeStruct((B, K, D), jnp.float32),
    )(x)
